# Initial kernel scaffold; baseline (speedup 1.0000x reference)
import jax
import jax.numpy as jnp
from jax.experimental import pallas as pl


def kernel(x, edge_index, W1, b1, W2, b2):
    raise NotImplementedError("write your pallas kernel here")


# speedup vs baseline: 42.2797x; 42.2797x over previous
import functools

import jax
import jax.numpy as jnp
from jax import lax
from jax.experimental import pallas as pl
from jax.experimental.pallas import tpu as pltpu
from jax.experimental.pallas import tpu_sc as plsc

N = 100000
E = 3200000

NC = 2
NS = 16
LANES = 128
MR = 8

NPAD = 100352
RPT = NPAD // NS
EPAD = 3211264
ROWS_E = EPAD // LANES
RW1 = ROWS_E // (NC * NS)
RW2 = ROWS_E // NS
NM1 = RW1 // MR
NM2 = RW2 // MR
ZR = 392

_mesh = plsc.VectorSubcoreMesh(core_axis_name="c", subcore_axis_name="s")
_f32 = jnp.float32
_SC_PARAMS = pltpu.CompilerParams(use_tc_tiling_on_sc=False)


def _zero_rows(buf, nrows, width):
    zv = jnp.zeros((16,), _f32)

    def body(i, _):
        if width == 1:
            buf[pl.ds(pl.multiple_of(i * 16, 16), 16)] = zv
        else:
            buf[i] = zv
        return ()

    lax.fori_loop(0, nrows, body, ())


@functools.partial(
    pl.kernel,
    out_type=[jax.ShapeDtypeStruct((NPAD,), _f32),
              jax.ShapeDtypeStruct((NPAD,), _f32)],
    mesh=_mesh,
    compiler_params=_SC_PARAMS,
    scratch_types=[
        pltpu.VMEM((MR, LANES), jnp.int32),
        pltpu.VMEM((LANES,), _f32),
        pltpu.VMEM((RPT,), _f32),
        pltpu.VMEM_SHARED((NPAD,), _f32),
        pltpu.SemaphoreType.DMA,
    ],
)
def _sc_deg(dst_hbm, out0_hbm, out1_hbm, dst_i, ones_v, zbuf, hist, sem):
    c = lax.axis_index("c")
    s = lax.axis_index("s")
    ov = jnp.ones((16,), _f32)
    for k in range(LANES // 16):
        ones_v[pl.ds(k * 16, 16)] = ov
    _zero_rows(zbuf, RPT // 16, 1)
    stripe = pl.multiple_of(s * RPT, 128)
    pltpu.sync_copy(zbuf, hist.at[pl.ds(stripe, RPT)])
    plsc.subcore_barrier()

    base = (c * NS + s) * RW1

    def mega(m, _):
        r0 = base + m * MR
        pltpu.sync_copy(dst_hbm.at[pl.ds(r0, MR)], dst_i)
        descs = [
            pltpu.async_copy(ones_v, hist.at[dst_i.at[j]], sem, add=True)
            for j in range(MR)
        ]
        for d in descs:
            d.wait()
        return ()

    lax.fori_loop(0, NM1, mega, ())
    plsc.subcore_barrier()

    @pl.when(c == 0)
    def _():
        pltpu.sync_copy(hist.at[pl.ds(stripe, RPT)],
                        out0_hbm.at[pl.ds(stripe, RPT)])

    @pl.when(c == 1)
    def _():
        pltpu.sync_copy(hist.at[pl.ds(stripe, RPT)],
                        out1_hbm.at[pl.ds(stripe, RPT)])


def _agg_loop(src_hbm, dst_hbm, table, acc, src_i, dst_i, rows, gsem, ssem,
              base, n_megas):
    def mega(m, _):
        r0 = base + m * MR
        pltpu.sync_copy(src_hbm.at[pl.ds(r0, MR)], src_i)
        pltpu.sync_copy(dst_hbm.at[pl.ds(r0, MR)], dst_i)
        gd = [
            pltpu.async_copy(table.at[src_i.at[j]],
                             rows.at[pl.ds(j * LANES, LANES)], gsem)
            for j in range(MR)
        ]
        for d in gd:
            d.wait()
        sd = [
            pltpu.async_copy(rows.at[pl.ds(j * LANES, LANES)],
                             acc.at[dst_i.at[j]], ssem, add=True)
            for j in range(MR)
        ]
        for d in sd:
            d.wait()
        return ()

    lax.fori_loop(0, n_megas, mega, ())


_AGG_SCRATCH = [
    pltpu.VMEM((MR, LANES), jnp.int32),
    pltpu.VMEM((MR, LANES), jnp.int32),
    pltpu.VMEM((MR * LANES, 16), _f32),
    pltpu.VMEM((ZR, 16), _f32),
    pltpu.VMEM_SHARED((NPAD, 16), _f32),
    pltpu.SemaphoreType.DMA,
    pltpu.SemaphoreType.DMA,
]


def _acc_init(acc, zbuf, s):
    _zero_rows(zbuf, ZR, 16)
    for k in range(RPT // ZR):
        off = pl.multiple_of(s * RPT + k * ZR, 8)
        pltpu.sync_copy(zbuf, acc.at[pl.ds(off, ZR)])
    plsc.subcore_barrier()


def _acc_dump(acc, out0, out1, c, s):
    plsc.subcore_barrier()
    stripe = pl.multiple_of(s * RPT, 128)

    @pl.when(c == 0)
    def _():
        pltpu.sync_copy(acc.at[pl.ds(stripe, RPT)],
                        out0.at[pl.ds(stripe, RPT)])

    @pl.when(c == 1)
    def _():
        pltpu.sync_copy(acc.at[pl.ds(stripe, RPT)],
                        out1.at[pl.ds(stripe, RPT)])


_AGG_OUT = [jax.ShapeDtypeStruct((NPAD, 16), _f32),
            jax.ShapeDtypeStruct((NPAD, 16), _f32)]


@functools.partial(
    pl.kernel,
    out_type=_AGG_OUT,
    mesh=_mesh,
    compiler_params=_SC_PARAMS,
    scratch_types=_AGG_SCRATCH,
)
def _sc_agg1(src_hbm, dst_hbm, g_hbm, out0_hbm, out1_hbm,
             src_i, dst_i, rows, zbuf, acc, gsem, ssem):
    c = lax.axis_index("c")
    s = lax.axis_index("s")
    _acc_init(acc, zbuf, s)
    base = (c * NS + s) * RW1
    _agg_loop(src_hbm, dst_hbm, g_hbm, acc, src_i, dst_i, rows, gsem, ssem,
              base, NM1)
    _acc_dump(acc, out0_hbm, out1_hbm, c, s)


@functools.partial(
    pl.kernel,
    out_type=_AGG_OUT,
    mesh=_mesh,
    compiler_params=_SC_PARAMS,
    scratch_types=_AGG_SCRATCH,
)
def _sc_agg2(src_hbm, dst_hbm, ga_hbm, gb_hbm, out0_hbm, out1_hbm,
             src_i, dst_i, rows, zbuf, acc, gsem, ssem):
    c = lax.axis_index("c")
    s = lax.axis_index("s")
    _acc_init(acc, zbuf, s)
    base = s * RW2

    @pl.when(c == 0)
    def _():
        _agg_loop(src_hbm, dst_hbm, ga_hbm, acc, src_i, dst_i, rows,
                  gsem, ssem, base, NM2)

    @pl.when(c == 1)
    def _():
        _agg_loop(src_hbm, dst_hbm, gb_hbm, acc, src_i, dst_i, rows,
                  gsem, ssem, base, NM2)

    _acc_dump(acc, out0_hbm, out1_hbm, c, s)


BR = 3136
_GRID = NPAD // BR


def _row_spec(width):
    return pl.BlockSpec((BR, width), lambda i: (i, 0))


def _whole_spec(shape):
    return pl.BlockSpec(shape, lambda i: (0,) * len(shape))


def _tca_body(x_ref, h0_ref, h1_ref, w_ref, g_ref, dinv_ref):
    deg = h0_ref[...] + h1_ref[...] + 1.0
    dinv = lax.rsqrt(deg)
    h = lax.dot_general(x_ref[...], w_ref[...], (((1,), (0,)), ((), ())),
                        preferred_element_type=_f32)
    g_ref[...] = h * dinv
    dinv_ref[...] = dinv


_tc_a = pl.pallas_call(
    _tca_body,
    grid=(_GRID,),
    in_specs=[_row_spec(3), _row_spec(1), _row_spec(1), _whole_spec((3, 16))],
    out_specs=[_row_spec(16), _row_spec(1)],
    out_shape=[jax.ShapeDtypeStruct((NPAD, 16), _f32),
               jax.ShapeDtypeStruct((NPAD, 1), _f32)],
)


def _tcb_body(p0_ref, p1_ref, g1_ref, dinv_ref, b1_ref, w_ref,
              ga_ref, gb_ref):
    dinv = dinv_ref[...]
    h = (p0_ref[...] + p1_ref[...] + g1_ref[...]) * dinv + b1_ref[...]
    h = jnp.maximum(h, 0.0)
    g2 = lax.dot_general(h, w_ref[...], (((1,), (0,)), ((), ())),
                         preferred_element_type=_f32) * dinv
    ga_ref[...] = g2[:, :16]
    gb_ref[...] = g2[:, 16:]


_tc_b = pl.pallas_call(
    _tcb_body,
    grid=(_GRID,),
    in_specs=[_row_spec(16), _row_spec(16), _row_spec(16), _row_spec(1),
              _whole_spec((1, 16)), _whole_spec((16, 32))],
    out_specs=[_row_spec(16), _row_spec(16)],
    out_shape=[jax.ShapeDtypeStruct((NPAD, 16), _f32),
               jax.ShapeDtypeStruct((NPAD, 16), _f32)],
)


def _tcc_body(qa_ref, qb_ref, ga_ref, gb_ref, dinv_ref, b2_ref, out_ref):
    dinv = dinv_ref[...]
    z = jnp.concatenate([qa_ref[...] + ga_ref[...],
                         qb_ref[...] + gb_ref[...]], axis=1)
    z = z * dinv + b2_ref[...]
    z = z - jnp.max(z, axis=1, keepdims=True)
    out_ref[...] = z - jnp.log(jnp.sum(jnp.exp(z), axis=1, keepdims=True))


_tc_c = pl.pallas_call(
    _tcc_body,
    grid=(_GRID,),
    in_specs=[_row_spec(16), _row_spec(16), _row_spec(16), _row_spec(16),
              _row_spec(1), _whole_spec((1, 32))],
    out_specs=_row_spec(32),
    out_shape=jax.ShapeDtypeStruct((NPAD, 32), _f32),
)


def kernel(x, edge_index, W1, b1, W2, b2):
    x = x.astype(_f32)
    src = edge_index[0]
    dst = edge_index[1]
    pad = jnp.full((EPAD - E,), N, jnp.int32)
    srcp = jnp.concatenate([src, pad]).reshape(ROWS_E, LANES)
    dstp = jnp.concatenate([dst, pad]).reshape(ROWS_E, LANES)
    xp = jnp.pad(x, ((0, NPAD - N), (0, 0)))

    h0, h1 = _sc_deg(dstp)
    g1, dinv = _tc_a(xp, h0[:, None], h1[:, None], W1)
    p0, p1 = _sc_agg1(srcp, dstp, g1)
    ga, gb = _tc_b(p0, p1, g1, dinv, b1[None, :], W2)
    qa, qb = _sc_agg2(srcp, dstp, ga, gb)
    out = _tc_c(qa, qb, ga, gb, dinv, b2[None, :])
    return out[:N]
